# baseline (speedup 1.0000x reference)
import functools

import jax
import jax.numpy as jnp
from jax import lax
from jax.experimental import pallas as pl
from jax.experimental.pallas import tpu as pltpu
from jax.experimental.pallas import tpu_sc as plsc

N = 10000
E = 320000
D = 128

NC = 2
NS = 16
NW = NC * NS
EPW = E // NW
K = 128
NBLK = 80
EPWP = NBLK * K
PHASES = 2
PBLK = NBLK // PHASES
N_PAD = 10240
ZCH = 128
NCHK_Z = N_PAD // ZCH
CH = 80
NCHK_D = N // CH
CPT_Z = (NCHK_Z + NS - 1) // NS
CPT_D = (NCHK_D + NS - 1) // NS


def _fill(buf, nrows, vec):
    def fill_row(i, _):
        for j in range(8):
            buf[i, pl.ds(j * 16, 16)] = vec
        return 0

    lax.fori_loop(0, nrows, fill_row, 0)


def _zero_acc(s, acc_sh, zbuf, zsem):
    def zero_chunk(t, _):
        chunk = jnp.minimum(s + t * NS, NCHK_Z - 1)
        pltpu.async_copy(zbuf, acc_sh.at[pl.ds(chunk * ZCH, ZCH)], zsem)
        return 0

    lax.fori_loop(0, CPT_Z, zero_chunk, 0)

    def zero_wait(t, _):
        pltpu.make_async_copy(zbuf, acc_sh.at[pl.ds(0, ZCH)], zsem).wait()
        return 0

    lax.fori_loop(0, CPT_Z, zero_wait, 0)


def _drain_acc(c, s, acc_sh, out_hbm, zsem):
    def drain_chunk(t, _):
        chunk = jnp.minimum(s + t * NS, NCHK_D - 1)
        r0 = chunk * CH
        pltpu.async_copy(acc_sh.at[pl.ds(r0, CH)],
                         out_hbm.at[c, pl.ds(r0, CH)], zsem)
        return 0

    lax.fori_loop(0, CPT_D, drain_chunk, 0)

    def drain_wait(t, _):
        pltpu.make_async_copy(acc_sh.at[pl.ds(0, CH)],
                              out_hbm.at[c, pl.ds(0, CH)], zsem).wait()
        return 0

    lax.fori_loop(0, CPT_D, drain_wait, 0)


def _sc_agg_body(h_hbm, src_hbm, dst_hbm, out_hbm, acc_sh, sidx, didx, rows,
                 gsem, ssem, zsem):
    c = lax.axis_index("c")
    s = lax.axis_index("s")
    wid = s * NC + c

    _fill(rows.at[1], ZCH, jnp.zeros((16,), jnp.float32))
    _zero_acc(s, acc_sh, rows.at[1], zsem)
    plsc.subcore_barrier()

    def fire_gather(b, i):
        pltpu.async_copy(h_hbm.at[sidx.at[b]], rows.at[i], gsem.at[i])

    def wait_gather(i):
        pltpu.make_async_copy(h_hbm.at[sidx.at[0]], rows.at[i],
                              gsem.at[i]).wait()

    def fire_scatter(b, i):
        pltpu.async_copy(rows.at[i], acc_sh.at[didx.at[b]], ssem.at[i],
                         add=True)

    def wait_scatter(i):
        pltpu.make_async_copy(rows.at[i], acc_sh.at[didx.at[0]],
                              ssem.at[i]).wait()

    for p in range(PHASES):
        pltpu.async_copy(src_hbm.at[wid, pl.ds(p * PBLK, PBLK)], sidx, zsem)
        pltpu.async_copy(dst_hbm.at[wid, pl.ds(p * PBLK, PBLK)], didx, zsem)
        pltpu.make_async_copy(src_hbm.at[wid, pl.ds(0, PBLK)], sidx,
                              zsem).wait()
        pltpu.make_async_copy(dst_hbm.at[wid, pl.ds(0, PBLK)], didx,
                              zsem).wait()

        fire_gather(0, 0)
        fire_gather(1, 1)

        def blk_body(q, _):
            for sl in range(2):
                b = q * 2 + sl
                wait_gather(sl)
                fire_scatter(b, sl)
                wait_scatter(sl)
                fire_gather(b + 2, sl)
            return 0

        lax.fori_loop(0, PBLK // 2 - 1, blk_body, 0)

        for sl in range(2):
            b = PBLK - 2 + sl
            wait_gather(sl)
            fire_scatter(b, sl)
            wait_scatter(sl)

    plsc.subcore_barrier()
    _drain_acc(c, s, acc_sh, out_hbm, zsem)


def _sc_deg_body(dst_hbm, out_hbm, acc_sh, didx, ones, zbuf, ssem, zsem):
    c = lax.axis_index("c")
    s = lax.axis_index("s")
    wid = s * NC + c

    _fill(zbuf, ZCH, jnp.zeros((16,), jnp.float32))
    _zero_acc(s, acc_sh, zbuf, zsem)
    _fill(ones, K, jnp.ones((16,), jnp.float32))
    plsc.subcore_barrier()

    def fire_scatter(b):
        pltpu.async_copy(ones, acc_sh.at[didx.at[b]], ssem, add=True)

    def wait_scatter():
        pltpu.make_async_copy(ones, acc_sh.at[didx.at[0]], ssem).wait()

    for p in range(PHASES):
        pltpu.async_copy(dst_hbm.at[wid, pl.ds(p * PBLK, PBLK)], didx, zsem)
        pltpu.make_async_copy(dst_hbm.at[wid, pl.ds(0, PBLK)], didx,
                              zsem).wait()

        for i in range(4):
            fire_scatter(i)

        def blk_body(q, _):
            fire_scatter(2 * q + 4)
            fire_scatter(2 * q + 5)
            wait_scatter()
            wait_scatter()
            return 0

        lax.fori_loop(0, (PBLK - 4) // 2, blk_body, 0)
        for i in range(4):
            wait_scatter()

    plsc.subcore_barrier()
    _drain_acc(c, s, acc_sh, out_hbm, zsem)


_MESH = plsc.VectorSubcoreMesh(core_axis_name="c", subcore_axis_name="s",
                               num_cores=NC, num_subcores=NS)


def _sc_aggregate(h, src3, dst3):
    kern = pl.kernel(
        _sc_agg_body,
        out_type=jax.ShapeDtypeStruct((NC, N, D), jnp.float32),
        mesh=_MESH,
        scratch_types=[
            pltpu.VMEM_SHARED((N_PAD, D), jnp.float32),
            pltpu.VMEM((PBLK, K), jnp.int32),
            pltpu.VMEM((PBLK, K), jnp.int32),
            pltpu.VMEM((2, K, D), jnp.float32),
            pltpu.SemaphoreType.DMA((2,)),
            pltpu.SemaphoreType.DMA((2,)),
            pltpu.SemaphoreType.DMA,
        ],
        name="sage_sc_agg",
    )
    return kern(h, src3, dst3)


def _sc_degree(dst3):
    kern = pl.kernel(
        _sc_deg_body,
        out_type=jax.ShapeDtypeStruct((NC, N, D), jnp.float32),
        mesh=_MESH,
        scratch_types=[
            pltpu.VMEM_SHARED((N_PAD, D), jnp.float32),
            pltpu.VMEM((PBLK, K), jnp.int32),
            pltpu.VMEM((K, D), jnp.float32),
            pltpu.VMEM((ZCH, D), jnp.float32),
            pltpu.SemaphoreType.DMA,
            pltpu.SemaphoreType.DMA,
        ],
        name="sage_sc_deg",
    )
    return kern(dst3)


def _tc_layer_body(relu, h_ref, a0_ref, a1_ref, d0_ref, d1_ref, ws_ref,
                   wn_ref, b_ref, o_ref):
    deg = jnp.maximum(d0_ref[:, 0:1] + d1_ref[:, 0:1], 1.0)
    mean = (a0_ref[...] + a1_ref[...]) / deg
    out = (jnp.dot(h_ref[...], ws_ref[...], preferred_element_type=jnp.float32)
           + jnp.dot(mean, wn_ref[...], preferred_element_type=jnp.float32)
           + b_ref[...])
    if relu:
        out = jnp.maximum(out, 0.0)
    o_ref[...] = out


def _tc_layer(h, A, degp, Ws, Wn, b, relu):
    F = Ws.shape[1]
    BN = 1000
    grid = (N // BN,)
    out = pl.pallas_call(
        functools.partial(_tc_layer_body, relu),
        grid=grid,
        in_specs=[
            pl.BlockSpec((BN, D), lambda i: (i, 0)),
            pl.BlockSpec((BN, D), lambda i: (i, 0)),
            pl.BlockSpec((BN, D), lambda i: (i, 0)),
            pl.BlockSpec((BN, D), lambda i: (i, 0)),
            pl.BlockSpec((BN, D), lambda i: (i, 0)),
            pl.BlockSpec((D, F), lambda i: (0, 0)),
            pl.BlockSpec((D, F), lambda i: (0, 0)),
            pl.BlockSpec((1, F), lambda i: (0, 0)),
        ],
        out_specs=pl.BlockSpec((BN, F), lambda i: (i, 0)),
        out_shape=jax.ShapeDtypeStruct((N, F), jnp.float32),
        name="sage_tc_layer",
    )(h, A[0], A[1], degp[0], degp[1], Ws, Wn, b.reshape(1, F))
    return out


def kernel(x, edge_index, Wn1, Ws1, b1, Wn2, Ws2, b2, Wn3, Ws3, b3):
    srcw = edge_index[0].reshape(NW, EPW)
    dstw = edge_index[1].reshape(NW, EPW)
    pad_n = EPWP - EPW
    junk = N + (jnp.arange(pad_n, dtype=jnp.int32) % (N_PAD - N))
    src3 = jnp.concatenate(
        [srcw, jnp.zeros((NW, pad_n), jnp.int32)], axis=1).reshape(NW, NBLK, K)
    dst3 = jnp.concatenate(
        [dstw, jnp.broadcast_to(junk, (NW, pad_n))], axis=1).reshape(
            NW, NBLK, K)
    degp = _sc_degree(dst3)
    A1 = _sc_aggregate(x, src3, dst3)
    h1 = _tc_layer(x, A1, degp, Ws1, Wn1, b1, relu=True)
    A2 = _sc_aggregate(h1, src3, dst3)
    h2 = _tc_layer(h1, A2, degp, Ws2, Wn2, b2, relu=True)
    A3 = _sc_aggregate(h2, src3, dst3)
    out = _tc_layer(h2, A3, degp, Ws3, Wn3, b3, relu=False)
    return out
